# baseline (speedup 1.0000x reference)
import functools

import jax
import jax.numpy as jnp
from jax.experimental import pallas as pl
from jax.experimental.pallas import tpu as pltpu


def _msg_block_kernel(e_ref, h_ref, we_ref, wh_ref, b_ref, o_ref, *, me, tn):
    j = pl.program_id(1)
    wt = we_ref[:me, :].astype(jnp.bfloat16)
    wb = wh_ref[me:, :].astype(jnp.bfloat16)
    bt = b_ref[:me]
    bb = b_ref[me:]
    e = e_ref[0, :, pl.ds(j * tn, tn)].astype(jnp.bfloat16)
    h = h_ref[0, :, pl.ds(j * tn, tn)].astype(jnp.bfloat16)
    top = jnp.dot(wt, e, preferred_element_type=jnp.float32)
    bot = jnp.dot(wb, h, preferred_element_type=jnp.float32)
    o_ref[0, :me] = jnp.maximum(top + bt, 0.0)
    o_ref[0, me:] = jnp.maximum(bot + bb, 0.0)


def kernel(e_vw, h_w, wk_e, wk_h, bk):
    B, Fe, N = e_vw.shape
    Fn = h_w.shape[1]
    Mout = wk_e.shape[0]
    me = Mout // 2

    nsplit = 4
    tn = N // nsplit
    grid = (B, nsplit)
    out_shape = jax.ShapeDtypeStruct((B, Mout, N), jnp.float32)
    in_specs = [
        pl.BlockSpec((1, Fe, N), lambda b, n: (b, 0, 0)),
        pl.BlockSpec((1, Fn, N), lambda b, n: (b, 0, 0)),
        pl.BlockSpec((Mout, Fe), lambda b, n: (0, 0)),
        pl.BlockSpec((Mout, Fn), lambda b, n: (0, 0)),
        pl.BlockSpec((Mout, 1), lambda b, n: (0, 0)),
    ]
    out_spec = pl.BlockSpec((1, Mout, tn), lambda b, n: (b, 0, n))

    flops = 2 * B * N * me * (Fe + Fn)
    bytes_accessed = B * N * 4 * (Fe + Fn + Mout)
    cost = pl.CostEstimate(flops=int(flops), transcendentals=0,
                           bytes_accessed=int(bytes_accessed))

    return pl.pallas_call(
        functools.partial(_msg_block_kernel, me=me, tn=tn),
        out_shape=out_shape,
        grid=grid,
        in_specs=in_specs,
        out_specs=out_spec,
        compiler_params=pltpu.CompilerParams(
            dimension_semantics=("parallel", "arbitrary")),
        cost_estimate=cost,
    )(e_vw, h_w, wk_e, wk_h, bk)


# speedup vs baseline: 1.5592x; 1.5592x over previous
import functools

import jax
import jax.numpy as jnp
from jax.experimental import pallas as pl
from jax.experimental.pallas import tpu as pltpu


def _msg_block_kernel(e_ref, h_ref, we_ref, wh_ref, b_ref, o_ref, *, me, tb):
    wt = we_ref[:me, :].astype(jnp.bfloat16)
    wb = wh_ref[me:, :].astype(jnp.bfloat16)
    bt = b_ref[:me]
    bb = b_ref[me:]
    for i in range(tb):
        e = e_ref[i].astype(jnp.bfloat16)
        h = h_ref[i].astype(jnp.bfloat16)
        top = jnp.dot(wt, e, preferred_element_type=jnp.float32)
        bot = jnp.dot(wb, h, preferred_element_type=jnp.float32)
        o_ref[i, :me] = jnp.maximum(top + bt, 0.0)
        o_ref[i, me:] = jnp.maximum(bot + bb, 0.0)


def kernel(e_vw, h_w, wk_e, wk_h, bk):
    B, Fe, N = e_vw.shape
    Fn = h_w.shape[1]
    Mout = wk_e.shape[0]
    me = Mout // 2

    tb = 1
    grid = (B // tb,)
    out_shape = jax.ShapeDtypeStruct((B, Mout, N), jnp.float32)
    in_specs = [
        pl.BlockSpec((tb, Fe, N), lambda b: (b, 0, 0)),
        pl.BlockSpec((tb, Fn, N), lambda b: (b, 0, 0)),
        pl.BlockSpec((Mout, Fe), lambda b: (0, 0)),
        pl.BlockSpec((Mout, Fn), lambda b: (0, 0)),
        pl.BlockSpec((Mout, 1), lambda b: (0, 0)),
    ]
    out_spec = pl.BlockSpec((tb, Mout, N), lambda b: (b, 0, 0))

    flops = 2 * B * N * me * (Fe + Fn)
    bytes_accessed = B * N * 4 * (Fe + Fn + Mout)
    cost = pl.CostEstimate(flops=int(flops), transcendentals=0,
                           bytes_accessed=int(bytes_accessed))

    return pl.pallas_call(
        functools.partial(_msg_block_kernel, me=me, tb=tb),
        out_shape=out_shape,
        grid=grid,
        in_specs=in_specs,
        out_specs=out_spec,
        compiler_params=pltpu.CompilerParams(
            dimension_semantics=("parallel",)),
        cost_estimate=cost,
    )(e_vw, h_w, wk_e, wk_h, bk)
